# baseline (speedup 1.0000x reference)
import functools

import jax
import jax.numpy as jnp
from jax import lax
from jax.experimental import pallas as pl
from jax.experimental.pallas import tpu as pltpu
from jax.experimental.pallas import tpu_sc as plsc


def _argmin_body(x_ref, c_ref, idx_ref, bestd_ref, besti_ref, *, nk, bk, kk):
    j = pl.program_id(1)
    x = x_ref[...]
    c = c_ref[...]
    dots = lax.dot_general(x.astype(jnp.bfloat16), c.astype(jnp.bfloat16),
                           (((1,), (1,)), ((), ())),
                           preferred_element_type=jnp.float32)
    x2 = jnp.sum(x * x, axis=1, keepdims=True)
    c2 = jnp.sum(c * c, axis=1)
    f = jnp.sqrt(jnp.maximum((x2 + c2[None, :]) - 2.0 * dots, 0.0))
    m = jnp.min(f, axis=1)
    iota = lax.broadcasted_iota(jnp.int32, f.shape, 1)
    cand = jnp.where(f == m[:, None], iota, kk)
    la = jnp.min(cand, axis=1).astype(jnp.int32) + j * bk

    @pl.when(j == 0)
    def _():
        bestd_ref[...] = m
        besti_ref[...] = la

    @pl.when(j > 0)
    def _():
        bd = bestd_ref[...]
        upd = m < bd
        bestd_ref[...] = jnp.where(upd, m, bd)
        besti_ref[...] = jnp.where(upd, la, besti_ref[...])

    @pl.when(j == nk - 1)
    def _():
        idx_ref[...] = besti_ref[...]


def _nearest_code_indices(x, codebook, bm, bk):
    m, d = x.shape
    k = codebook.shape[0]
    nm, nk = m // bm, k // bk
    return pl.pallas_call(
        functools.partial(_argmin_body, nk=nk, bk=bk, kk=k),
        grid=(nm, nk),
        in_specs=[
            pl.BlockSpec((bm, d), lambda i, j: (i, 0)),
            pl.BlockSpec((bk, d), lambda i, j: (j, 0)),
        ],
        out_specs=pl.BlockSpec((bm,), lambda i, j: (i,)),
        out_shape=jax.ShapeDtypeStruct((m,), jnp.int32),
        scratch_shapes=[
            pltpu.VMEM((bm,), jnp.float32),
            pltpu.VMEM((bm,), jnp.int32),
        ],
        compiler_params=pltpu.CompilerParams(
            dimension_semantics=("parallel", "arbitrary")),
    )(x, codebook)


def _sc_gather(codebook, idx):
    info = plsc.get_sparse_core_info()
    nc, ns = info.num_cores, info.num_subcores
    nw = nc * ns
    m = idx.shape[0]
    d = codebook.shape[1]
    bpw = m // nw
    mesh = plsc.VectorSubcoreMesh(core_axis_name="c", subcore_axis_name="s")

    @functools.partial(
        pl.kernel, mesh=mesh,
        out_type=jax.ShapeDtypeStruct((m, d), jnp.float32),
        scratch_types=[
            pltpu.VMEM((bpw,), jnp.int32),
            pltpu.VMEM((bpw, d), jnp.float32),
            pltpu.SemaphoreType.DMA,
        ],
        compiler_params=pltpu.CompilerParams(use_tc_tiling_on_sc=False),
    )
    def gk(table_hbm, idx_hbm, out_hbm, idx_v, rows_v, sem):
        wid = lax.axis_index("s") * nc + lax.axis_index("c")
        base = wid * bpw
        pltpu.sync_copy(idx_hbm.at[pl.ds(base, bpw)], idx_v)
        pltpu.async_copy(table_hbm.at[idx_v], rows_v, sem).wait()
        pltpu.sync_copy(rows_v, out_hbm.at[pl.ds(base, bpw)])

    return gk(codebook, idx)


def kernel(x_in, codebook):
    b, n, d = x_in.shape
    x = x_in.reshape(b * n, d)
    idx = _nearest_code_indices(x, codebook, bm=1024, bk=2048)
    quant = _sc_gather(codebook, idx)
    return quant.reshape(b, n, d)
